# baseline (speedup 1.0000x reference)
import functools

import jax
import jax.numpy as jnp
import numpy as np
from jax import lax
from jax.experimental import pallas as pl
from jax.experimental.pallas import tpu as pltpu
from jax.experimental.pallas import tpu_sc as plsc

_N = 10000
_E = 320000
_D1 = 128
_D2 = 16

_NCORES = 2
_NSUB = 16
_NTILES = _NCORES * _NSUB
_B = 80
_ROUNDS = 128
_CHUNK = 16
_NCHUNKS = _ROUNDS // _CHUNK
_E_PER_TILE = _ROUNDS * _B
_E_PAD = _E_PER_TILE * _NTILES
_ROWS_PER_SUB = 632
_N_PAD = _ROWS_PER_SUB * _NSUB

_PERM = np.empty(_D1, np.int32)
for _m in range(_D1 // 32):
    for _j in range(16):
        _PERM[32 * _m + _j] = 32 * _m + 2 * _j
        _PERM[32 * _m + 16 + _j] = 32 * _m + 2 * _j + 1


def _mm_body(x_ref, w_ref, o_ref):
    r = jnp.dot(x_ref[...], w_ref[...],
                preferred_element_type=jnp.float32,
                precision=lax.Precision.HIGHEST)
    o_ref[...] = r.astype(o_ref.dtype)


def _matmul(x, w, out_dtype=jnp.float32):
    return pl.pallas_call(
        _mm_body,
        out_shape=jax.ShapeDtypeStruct((x.shape[0], w.shape[1]), out_dtype),
    )(x, w)


def _relu_mm_body(p_ref, w_ref, o_ref):
    h = jnp.maximum(p_ref[0] + p_ref[1], 0.0)
    o_ref[...] = jnp.dot(h, w_ref[...],
                         preferred_element_type=jnp.float32,
                         precision=lax.Precision.HIGHEST)


def _relu_mm(p, w):
    return pl.pallas_call(
        _relu_mm_body,
        out_shape=jax.ShapeDtypeStruct((p.shape[1], w.shape[1]), jnp.float32),
    )(p, w)


def _add2_body(q_ref, o_ref):
    o_ref[...] = q_ref[0, :_N, :] + q_ref[1, :_N, :]


def _add2(q):
    return pl.pallas_call(
        _add2_body,
        out_shape=jax.ShapeDtypeStruct((_N, q.shape[2]), jnp.float32),
    )(q)


_GDN = lax.GatherDimensionNumbers(
    offset_dims=(), collapsed_slice_dims=(0,), start_index_map=(0,))


def _bcast_lane(v, i):
    idx = jnp.full((16, 1), i, jnp.int32)
    return lax.gather(v, idx, _GDN, (1,),
                      mode=lax.GatherScatterMode.PROMISE_IN_BOUNDS)


def _make_agg(d, bf16_gather):
    mesh = plsc.VectorSubcoreMesh(core_axis_name="c", subcore_axis_name="s")
    gdtype = jnp.bfloat16 if bf16_gather else jnp.float32

    @functools.partial(
        pl.kernel,
        out_type=jax.ShapeDtypeStruct((_NCORES, _N_PAD, d), jnp.float32),
        mesh=mesh,
        scratch_types=[
            pltpu.VMEM_SHARED((_N_PAD, d), jnp.float32),
            pltpu.VMEM((_CHUNK, _B), jnp.int32),
            pltpu.VMEM((_CHUNK, _B), jnp.int32),
            pltpu.VMEM((_CHUNK, _B), jnp.float32),
            pltpu.VMEM((_B, d), gdtype),
            pltpu.VMEM((_B, d), gdtype),
            pltpu.VMEM((_B, d), jnp.float32),
            pltpu.VMEM((_B, d), jnp.float32),
            pltpu.SemaphoreType.DMA,
            pltpu.SemaphoreType.DMA,
            pltpu.SemaphoreType.DMA,
            pltpu.SemaphoreType.DMA,
        ],
        compiler_params=pltpu.CompilerParams(
            use_tc_tiling_on_sc=False, needs_layout_passes=False),
    )
    def agg(y_hbm, src_hbm, dst_hbm, ew_hbm, out_hbm,
            acc, src_v, dst_v, ew_v, bf0, bf1, rf0, rf1, g0, g1, s0, s1):
        c = lax.axis_index("c")
        s = lax.axis_index("s")
        @pl.loop(0, _B)
        def _zrow(r):
            for j in range(d // 16):
                rf0[r, pl.ds(j * 16, 16)] = jnp.zeros((16,), jnp.float32)

        row0 = s * _ROWS_PER_SUB
        nfull = _ROWS_PER_SUB // _B
        rem = _ROWS_PER_SUB - nfull * _B
        for k in range(nfull):
            pltpu.sync_copy(rf0, acc.at[pl.ds(row0 + k * _B, _B)])
        pltpu.sync_copy(rf0.at[pl.ds(0, rem)],
                        acc.at[pl.ds(row0 + nfull * _B, rem)])
        plsc.subcore_barrier()

        blk0 = (c * _NSUB + s) * _ROUNDS

        def start_gather(r, rows, sem):
            pltpu.async_copy(y_hbm.at[src_v.at[r]], rows, sem)

        def wait_gather(r, rows, sem):
            pltpu.make_async_copy(y_hbm.at[src_v.at[r]], rows, sem).wait()

        def start_scatter(r, rows, sem):
            pltpu.async_copy(rows, acc.at[dst_v.at[r]], sem, add=True)

        def wait_scatter(r, rows, sem):
            pltpu.make_async_copy(rows, acc.at[dst_v.at[r]], sem).wait()

        def scale(r, rows_in, rows_out):
            @pl.loop(0, _B // 16)
            def _group(g):
                ew16 = ew_v[r, pl.ds(g * 16, 16)]
                for i in range(16):
                    w = _bcast_lane(ew16, i)
                    e = g * 16 + i
                    if bf16_gather:
                        for m in range(d // 32):
                            v = rows_in[e, pl.ds(32 * m, 32)]
                            a, b = plsc.unpack(
                                v, format=plsc.PackFormat.INTERLEAVED,
                                preferred_element_type=jnp.float32)
                            rows_out[e, pl.ds(32 * m, 16)] = a * w
                            rows_out[e, pl.ds(32 * m + 16, 16)] = b * w
                    else:
                        for j in range(d // 16):
                            sl = pl.ds(j * 16, 16)
                            rows_out[e, sl] = rows_in[e, sl] * w

        @pl.loop(0, _NCHUNKS)
        def _chunk(ch):
            blk = blk0 + ch * _CHUNK
            pltpu.sync_copy(src_hbm.at[pl.ds(blk, _CHUNK)], src_v)
            pltpu.sync_copy(dst_hbm.at[pl.ds(blk, _CHUNK)], dst_v)
            pltpu.sync_copy(ew_hbm.at[pl.ds(blk, _CHUNK)], ew_v)
            start_gather(0, bf0, g0)

            @pl.loop(0, _CHUNK // 2)
            def _pair(t):
                ra = 2 * t
                rb = 2 * t + 1

                start_gather(rb, bf1, g1)
                wait_gather(ra, bf0, g0)

                @pl.when(t > 0)
                def _():
                    wait_scatter(ra, rf0, s0)
                scale(ra, bf0, rf0)
                start_scatter(ra, rf0, s0)

                @pl.when(t < _CHUNK // 2 - 1)
                def _():
                    start_gather(2 * t + 2, bf0, g0)
                wait_gather(rb, bf1, g1)

                @pl.when(t > 0)
                def _():
                    wait_scatter(rb, rf1, s1)
                scale(rb, bf1, rf1)
                start_scatter(rb, rf1, s1)

            wait_scatter(_CHUNK - 2, rf0, s0)
            wait_scatter(_CHUNK - 1, rf1, s1)

        plsc.subcore_barrier()
        pltpu.sync_copy(acc.at[pl.ds(row0, _ROWS_PER_SUB)],
                        out_hbm.at[c, pl.ds(row0, _ROWS_PER_SUB)])

    return agg


_agg128 = _make_agg(_D1, bf16_gather=True)
_agg16 = _make_agg(_D2, bf16_gather=False)


def kernel(x, edge_index, edge_weight, W1, W2):
    src = edge_index[0].astype(jnp.int32)
    dst = edge_index[1].astype(jnp.int32)
    ew = edge_weight.astype(jnp.float32)

    pad = _E_PAD - _E
    pad_idx = jnp.arange(pad, dtype=jnp.int32) % _N
    src_p = jnp.concatenate([src, pad_idx]).reshape(_E_PAD // _B, _B)
    dst_p = jnp.concatenate([dst, pad_idx]).reshape(_E_PAD // _B, _B)
    ew_p = jnp.concatenate(
        [ew, jnp.zeros((pad,), jnp.float32)]).reshape(_E_PAD // _B, _B)

    w2tp = W2.T[jnp.asarray(_PERM)]

    y1 = _matmul(x, W1.T, out_dtype=jnp.bfloat16)
    p = _agg128(y1, src_p, dst_p, ew_p)
    h2 = _relu_mm(p, w2tp)
    q = _agg16(h2, src_p, dst_p, ew_p)
    return _add2(q)


# speedup vs baseline: 1.4360x; 1.4360x over previous
import functools

import jax
import jax.numpy as jnp
from jax import lax
from jax.experimental import pallas as pl
from jax.experimental.pallas import tpu as pltpu
from jax.experimental.pallas import tpu_sc as plsc

_N = 10000
_E = 320000
_D1 = 128
_D2 = 16

_NCORES = 2
_NSUB = 16
_NTILES = _NCORES * _NSUB
_B = 128
_H = _B // 2
_ROUNDS = 80
_CHUNK = 16
_NCHUNKS = _ROUNDS // _CHUNK
_E_PER_TILE = _ROUNDS * _B
_E_PAD = _E_PER_TILE * _NTILES
_ROWS_PER_SUB = 632
_N_PAD = _ROWS_PER_SUB * _NSUB


def _mm_body(x_ref, w_ref, o_ref):
    o_ref[...] = jnp.dot(x_ref[...], w_ref[...],
                         preferred_element_type=jnp.float32,
                         precision=lax.Precision.HIGHEST)


def _matmul(x, w):
    return pl.pallas_call(
        _mm_body,
        out_shape=jax.ShapeDtypeStruct((x.shape[0], w.shape[1]), jnp.float32),
    )(x, w)


def _relu_mm_body(p_ref, w_ref, o_ref):
    h = jnp.maximum(p_ref[0] + p_ref[1], 0.0)
    o_ref[...] = jnp.dot(h, w_ref[...],
                         preferred_element_type=jnp.float32,
                         precision=lax.Precision.HIGHEST)


def _relu_mm(p, w):
    return pl.pallas_call(
        _relu_mm_body,
        out_shape=jax.ShapeDtypeStruct((p.shape[1], w.shape[1]), jnp.float32),
    )(p, w)


def _add2_body(q_ref, o_ref):
    o_ref[...] = q_ref[0, :_N, :] + q_ref[1, :_N, :]


def _add2(q):
    return pl.pallas_call(
        _add2_body,
        out_shape=jax.ShapeDtypeStruct((_N, q.shape[2]), jnp.float32),
    )(q)


_GDN = lax.GatherDimensionNumbers(
    offset_dims=(), collapsed_slice_dims=(0,), start_index_map=(0,))


def _bcast_lane(v, i):
    idx = jnp.full((16, 1), i, jnp.int32)
    return lax.gather(v, idx, _GDN, (1,),
                      mode=lax.GatherScatterMode.PROMISE_IN_BOUNDS)


def _make_agg(d):
    mesh = plsc.VectorSubcoreMesh(core_axis_name="c", subcore_axis_name="s")

    @functools.partial(
        pl.kernel,
        out_type=jax.ShapeDtypeStruct((_NCORES, _N_PAD, d), jnp.float32),
        mesh=mesh,
        scratch_types=[
            pltpu.VMEM_SHARED((_N_PAD, d), jnp.float32),
            pltpu.VMEM((2 * _CHUNK, _H), jnp.int32),
            pltpu.VMEM((2 * _CHUNK, _H), jnp.int32),
            pltpu.VMEM((_CHUNK, _B), jnp.float32),
            pltpu.VMEM((_B, d), jnp.float32),
            pltpu.VMEM((_B, d), jnp.float32),
            pltpu.SemaphoreType.DMA,
            pltpu.SemaphoreType.DMA,
            pltpu.SemaphoreType.DMA,
            pltpu.SemaphoreType.DMA,
        ],
        compiler_params=pltpu.CompilerParams(
            use_tc_tiling_on_sc=(d % 128 == 0)),
    )
    def agg(y_hbm, src_hbm, dst_hbm, ew_hbm, out_hbm,
            acc, src_v, dst_v, ew_v, rows0, rows1, g0, g1, s0, s1):
        c = lax.axis_index("c")
        s = lax.axis_index("s")
        @pl.loop(0, _B)
        def _zrow(r):
            for j in range(d // 16):
                rows0[r, pl.ds(j * 16, 16)] = jnp.zeros((16,), jnp.float32)

        row0 = s * _ROWS_PER_SUB
        nfull = _ROWS_PER_SUB // _B
        rem = _ROWS_PER_SUB - nfull * _B
        for k in range(nfull):
            pltpu.sync_copy(rows0, acc.at[pl.ds(row0 + k * _B, _B)])
        pltpu.sync_copy(rows0.at[pl.ds(0, rem)],
                        acc.at[pl.ds(row0 + nfull * _B, rem)])
        plsc.subcore_barrier()

        blk0 = (c * _NSUB + s) * _ROUNDS

        def start_gather(r, rows, sem):
            pltpu.async_copy(y_hbm.at[src_v.at[2 * r]],
                             rows.at[pl.ds(0, _H)], sem)
            pltpu.async_copy(y_hbm.at[src_v.at[2 * r + 1]],
                             rows.at[pl.ds(_H, _H)], sem)

        def wait_gather(r, rows, sem):
            pltpu.make_async_copy(y_hbm.at[src_v.at[2 * r]],
                                  rows.at[pl.ds(0, _H)], sem).wait()
            pltpu.make_async_copy(y_hbm.at[src_v.at[2 * r + 1]],
                                  rows.at[pl.ds(_H, _H)], sem).wait()

        def start_scatter(r, rows, sem):
            pltpu.async_copy(rows.at[pl.ds(0, _H)],
                             acc.at[dst_v.at[2 * r]], sem, add=True)
            pltpu.async_copy(rows.at[pl.ds(_H, _H)],
                             acc.at[dst_v.at[2 * r + 1]], sem, add=True)

        def wait_scatter(r, rows, sem):
            pltpu.make_async_copy(rows.at[pl.ds(0, _H)],
                                  acc.at[dst_v.at[2 * r]], sem).wait()
            pltpu.make_async_copy(rows.at[pl.ds(_H, _H)],
                                  acc.at[dst_v.at[2 * r + 1]], sem).wait()

        def scale(r, rows):
            @pl.loop(0, _B // 16)
            def _group(g):
                ew16 = ew_v[r, pl.ds(g * 16, 16)]
                for i in range(16):
                    w = _bcast_lane(ew16, i)
                    e = g * 16 + i
                    for j in range(d // 16):
                        sl = pl.ds(j * 16, 16)
                        rows[e, sl] = rows[e, sl] * w

        @pl.loop(0, _NCHUNKS)
        def _chunk(ch):
            blk = blk0 + ch * _CHUNK
            pltpu.sync_copy(src_hbm.at[pl.ds(2 * blk, 2 * _CHUNK)], src_v)
            pltpu.sync_copy(dst_hbm.at[pl.ds(2 * blk, 2 * _CHUNK)], dst_v)
            pltpu.sync_copy(ew_hbm.at[pl.ds(blk, _CHUNK)], ew_v)
            start_gather(0, rows0, g0)

            @pl.loop(0, _CHUNK // 2)
            def _pair(t):
                ra = 2 * t
                rb = 2 * t + 1

                @pl.when(t > 0)
                def _():
                    wait_scatter(rb, rows1, s1)
                start_gather(rb, rows1, g1)
                wait_gather(ra, rows0, g0)
                scale(ra, rows0)
                start_scatter(ra, rows0, s0)
                wait_gather(rb, rows1, g1)
                scale(rb, rows1)
                start_scatter(rb, rows1, s1)
                wait_scatter(ra, rows0, s0)

                @pl.when(t < _CHUNK // 2 - 1)
                def _():
                    start_gather(2 * t + 2, rows0, g0)

            wait_scatter(_CHUNK - 1, rows1, s1)

        plsc.subcore_barrier()
        pltpu.sync_copy(acc.at[pl.ds(row0, _ROWS_PER_SUB)],
                        out_hbm.at[c, pl.ds(row0, _ROWS_PER_SUB)])

    return agg


_agg128 = _make_agg(_D1)
_agg16 = _make_agg(_D2)


def kernel(x, edge_index, edge_weight, W1, W2):
    src = edge_index[0].astype(jnp.int32)
    dst = edge_index[1].astype(jnp.int32)
    ew = edge_weight.astype(jnp.float32)

    pad = _E_PAD - _E
    pad_idx = jnp.arange(pad, dtype=jnp.int32) % _N
    src_p = jnp.concatenate([src, pad_idx]).reshape(_E_PAD // _H, _H)
    dst_p = jnp.concatenate([dst, pad_idx]).reshape(_E_PAD // _H, _H)
    ew_p = jnp.concatenate(
        [ew, jnp.zeros((pad,), jnp.float32)]).reshape(_E_PAD // _B, _B)

    y1 = _matmul(x, W1.T)
    p = _agg128(y1, src_p, dst_p, ew_p)
    h2 = _relu_mm(p, W2.T)
    q = _agg16(h2, src_p, dst_p, ew_p)
    return _add2(q)


# speedup vs baseline: 1.5431x; 1.0746x over previous
import functools

import jax
import jax.numpy as jnp
from jax import lax
from jax.experimental import pallas as pl
from jax.experimental.pallas import tpu as pltpu
from jax.experimental.pallas import tpu_sc as plsc

_N = 10000
_E = 320000
_D1 = 128
_D2 = 16

_NCORES = 2
_NSUB = 16
_NTILES = _NCORES * _NSUB
_B = 128
_H = _B // 2
_ROUNDS = 80
_CHUNK = 16
_NCHUNKS = _ROUNDS // _CHUNK
_E_PER_TILE = _ROUNDS * _B
_E_PAD = _E_PER_TILE * _NTILES
_ROWS_PER_SUB = 632
_N_PAD = _ROWS_PER_SUB * _NSUB


def _mm_body(x_ref, w_ref, o_ref):
    o_ref[...] = jnp.dot(x_ref[...], w_ref[...],
                         preferred_element_type=jnp.float32,
                         precision=lax.Precision.HIGHEST)


def _matmul(x, w):
    return pl.pallas_call(
        _mm_body,
        out_shape=jax.ShapeDtypeStruct((x.shape[0], w.shape[1]), jnp.float32),
    )(x, w)


def _relu_mm_body(p_ref, w_ref, o_ref):
    h = jnp.maximum(p_ref[0] + p_ref[1], 0.0)
    o_ref[...] = jnp.dot(h, w_ref[...],
                         preferred_element_type=jnp.float32,
                         precision=lax.Precision.HIGHEST)


def _relu_mm(p, w):
    return pl.pallas_call(
        _relu_mm_body,
        out_shape=jax.ShapeDtypeStruct((p.shape[1], w.shape[1]), jnp.float32),
    )(p, w)


def _add2_body(q_ref, o_ref):
    o_ref[...] = q_ref[0, :_N, :] + q_ref[1, :_N, :]


def _add2(q):
    return pl.pallas_call(
        _add2_body,
        out_shape=jax.ShapeDtypeStruct((_N, q.shape[2]), jnp.float32),
    )(q)


_GDN = lax.GatherDimensionNumbers(
    offset_dims=(), collapsed_slice_dims=(0,), start_index_map=(0,))


def _bcast_lane(v, i):
    idx = jnp.full((16, 1), i, jnp.int32)
    return lax.gather(v, idx, _GDN, (1,),
                      mode=lax.GatherScatterMode.PROMISE_IN_BOUNDS)


def _make_agg(d, stage_y=False):
    mesh = plsc.VectorSubcoreMesh(core_axis_name="c", subcore_axis_name="s")
    scratch = [
        pltpu.VMEM_SHARED((_N_PAD, d), jnp.float32),
        pltpu.VMEM((2 * _CHUNK, _H), jnp.int32),
        pltpu.VMEM((2 * _CHUNK, _H), jnp.int32),
        pltpu.VMEM((_CHUNK, _B), jnp.float32),
        pltpu.VMEM((_B, d), jnp.float32),
        pltpu.VMEM((_B, d), jnp.float32),
        pltpu.SemaphoreType.DMA,
        pltpu.SemaphoreType.DMA,
        pltpu.SemaphoreType.DMA,
        pltpu.SemaphoreType.DMA,
    ]
    if stage_y:
        scratch.append(pltpu.VMEM_SHARED((_N_PAD, d), jnp.float32))

    @functools.partial(
        pl.kernel,
        out_type=jax.ShapeDtypeStruct((_NCORES, _N_PAD, d), jnp.float32),
        mesh=mesh,
        scratch_types=scratch,
        compiler_params=pltpu.CompilerParams(
            use_tc_tiling_on_sc=(d % 128 == 0)),
    )
    def agg(y_hbm, src_hbm, dst_hbm, ew_hbm, out_hbm,
            acc, src_v, dst_v, ew_v, rows0, rows1, g0, g1, s0, s1,
            *maybe_ysh):
        c = lax.axis_index("c")
        s = lax.axis_index("s")
        @pl.loop(0, _B)
        def _zrow(r):
            for j in range(d // 16):
                rows0[r, pl.ds(j * 16, 16)] = jnp.zeros((16,), jnp.float32)

        row0 = s * _ROWS_PER_SUB
        nfull = _ROWS_PER_SUB // _B
        rem = _ROWS_PER_SUB - nfull * _B
        for k in range(nfull):
            pltpu.sync_copy(rows0, acc.at[pl.ds(row0 + k * _B, _B)])
        pltpu.sync_copy(rows0.at[pl.ds(0, rem)],
                        acc.at[pl.ds(row0 + nfull * _B, rem)])
        if stage_y:
            y_src = maybe_ysh[0]
            pltpu.sync_copy(y_hbm.at[pl.ds(row0, _ROWS_PER_SUB)],
                            y_src.at[pl.ds(row0, _ROWS_PER_SUB)])
        else:
            y_src = y_hbm
        plsc.subcore_barrier()

        blk0 = (c * _NSUB + s) * _ROUNDS

        def start_gather(r, rows, sem):
            pltpu.async_copy(y_src.at[src_v.at[2 * r]],
                             rows.at[pl.ds(0, _H)], sem)
            pltpu.async_copy(y_src.at[src_v.at[2 * r + 1]],
                             rows.at[pl.ds(_H, _H)], sem)

        def wait_gather(r, rows, sem):
            pltpu.make_async_copy(y_src.at[src_v.at[2 * r]],
                                  rows.at[pl.ds(0, _H)], sem).wait()
            pltpu.make_async_copy(y_src.at[src_v.at[2 * r + 1]],
                                  rows.at[pl.ds(_H, _H)], sem).wait()

        def start_scatter(r, rows, sem):
            pltpu.async_copy(rows.at[pl.ds(0, _H)],
                             acc.at[dst_v.at[2 * r]], sem, add=True)
            pltpu.async_copy(rows.at[pl.ds(_H, _H)],
                             acc.at[dst_v.at[2 * r + 1]], sem, add=True)

        def wait_scatter(r, rows, sem):
            pltpu.make_async_copy(rows.at[pl.ds(0, _H)],
                                  acc.at[dst_v.at[2 * r]], sem).wait()
            pltpu.make_async_copy(rows.at[pl.ds(_H, _H)],
                                  acc.at[dst_v.at[2 * r + 1]], sem).wait()

        def scale(r, rows):
            @pl.loop(0, _B // 16)
            def _group(g):
                ew16 = ew_v[r, pl.ds(g * 16, 16)]
                for i in range(16):
                    w = _bcast_lane(ew16, i)
                    e = g * 16 + i
                    for j in range(d // 16):
                        sl = pl.ds(j * 16, 16)
                        rows[e, sl] = rows[e, sl] * w

        @pl.loop(0, _NCHUNKS)
        def _chunk(ch):
            blk = blk0 + ch * _CHUNK
            pltpu.sync_copy(src_hbm.at[pl.ds(2 * blk, 2 * _CHUNK)], src_v)
            pltpu.sync_copy(dst_hbm.at[pl.ds(2 * blk, 2 * _CHUNK)], dst_v)
            pltpu.sync_copy(ew_hbm.at[pl.ds(blk, _CHUNK)], ew_v)
            start_gather(0, rows0, g0)

            @pl.loop(0, _CHUNK // 2)
            def _pair(t):
                ra = 2 * t
                rb = 2 * t + 1

                @pl.when(t > 0)
                def _():
                    wait_scatter(rb, rows1, s1)
                start_gather(rb, rows1, g1)
                wait_gather(ra, rows0, g0)
                scale(ra, rows0)
                start_scatter(ra, rows0, s0)
                wait_gather(rb, rows1, g1)
                scale(rb, rows1)
                start_scatter(rb, rows1, s1)
                wait_scatter(ra, rows0, s0)

                @pl.when(t < _CHUNK // 2 - 1)
                def _():
                    start_gather(2 * t + 2, rows0, g0)

            wait_scatter(_CHUNK - 1, rows1, s1)

        plsc.subcore_barrier()
        pltpu.sync_copy(acc.at[pl.ds(row0, _ROWS_PER_SUB)],
                        out_hbm.at[c, pl.ds(row0, _ROWS_PER_SUB)])

    return agg


_agg128 = _make_agg(_D1)
_agg16 = _make_agg(_D2, stage_y=True)


def kernel(x, edge_index, edge_weight, W1, W2):
    src = edge_index[0].astype(jnp.int32)
    dst = edge_index[1].astype(jnp.int32)
    ew = edge_weight.astype(jnp.float32)

    pad = _E_PAD - _E
    pad_idx = jnp.arange(pad, dtype=jnp.int32) % _N
    src_p = jnp.concatenate([src, pad_idx]).reshape(_E_PAD // _H, _H)
    dst_p = jnp.concatenate([dst, pad_idx]).reshape(_E_PAD // _H, _H)
    ew_p = jnp.concatenate(
        [ew, jnp.zeros((pad,), jnp.float32)]).reshape(_E_PAD // _B, _B)

    y1 = _matmul(x, W1.T)
    p = _agg128(y1, src_p, dst_p, ew_p)
    h2 = _relu_mm(p, W2.T)
    q = _agg16(h2, src_p, dst_p, ew_p)
    return _add2(q)
